# baseline (speedup 1.0000x reference)
import functools

import jax
import jax.numpy as jnp
from jax import lax
from jax.experimental import pallas as pl
from jax.experimental.pallas import tpu as pltpu
from jax.experimental.pallas import tpu_sc as plsc


def _make_gather(V, D, B):
    info = plsc.get_sparse_core_info()
    NC, NS = info.num_cores, info.num_subcores
    NW = NC * NS
    assert B % (8 * NW) == 0 and V % 2 == 0
    b_per_w = B // NW
    C = 128
    n_chunks = b_per_w // C
    mesh = plsc.VectorSubcoreMesh(core_axis_name="c", subcore_axis_name="s")

    @functools.partial(
        pl.kernel,
        mesh=mesh,
        out_type=jax.ShapeDtypeStruct((B // 2, 2 * D), jnp.float32),
        scratch_types=[
            pltpu.VMEM((b_per_w,), jnp.int32),
            pltpu.VMEM((b_per_w,), jnp.int32),
            pltpu.VMEM((C, 2 * D), jnp.float32),
            pltpu.VMEM((C, 2 * D), jnp.float32),
            pltpu.VMEM((C // 2, 2 * D), jnp.float32),
            pltpu.SemaphoreType.DMA,
            pltpu.SemaphoreType.DMA,
        ],
    )
    def gather_kernel(y_hbm, table_hbm, out_hbm, y_v, t_v, g0, g1, rows_v,
                      sem0, sem1):
        wid = lax.axis_index("s") * NC + lax.axis_index("c")
        base = wid * b_per_w
        pltpu.sync_copy(y_hbm.at[pl.ds(base, b_per_w)], y_v)

        @pl.loop(0, b_per_w // 16, unroll=4)
        def _(k):
            t_v[pl.ds(k * 16, 16)] = y_v[pl.ds(k * 16, 16)] >> 1

        def fire(c, g, sm):
            pltpu.async_copy(table_hbm.at[t_v.at[pl.ds(c * C, C)]], g, sm)

        def drain_extract_store(c, g, sm):
            pltpu.make_async_copy(table_hbm.at[pl.ds(0, C)], g, sm).wait()

            @pl.loop(0, C // 16)
            def _(k):
                p = y_v[pl.ds(c * C + k * 16, 16)] & 1
                for j in range(16):
                    half = p[j] * D
                    for u in range(D // 16):
                        rows_v[k * 8 + j // 2,
                               pl.ds((j % 2) * D + u * 16, 16)] = (
                            g[k * 16 + j, pl.ds(half + u * 16, 16)]
                        )

            off = pl.multiple_of(base // 2 + c * (C // 2), 8)
            pltpu.sync_copy(rows_v, out_hbm.at[pl.ds(off, C // 2)])

        fire(0, g0, sem0)

        @pl.loop(0, n_chunks // 2)
        def _(m):
            c0 = 2 * m
            fire(c0 + 1, g1, sem1)
            drain_extract_store(c0, g0, sem0)

            @pl.when(c0 + 2 < n_chunks)
            def _():
                fire(c0 + 2, g0, sem0)

            drain_extract_store(c0 + 1, g1, sem1)

    return gather_kernel


@jax.jit
def kernel(y, table):
    B, = y.shape
    V, D = table.shape
    table2 = table.reshape(V // 2, 2 * D)
    out2 = _make_gather(V, D, B)(y.astype(jnp.int32), table2)
    return out2.reshape(B, D)


# speedup vs baseline: 1.0099x; 1.0099x over previous
import functools

import jax
import jax.numpy as jnp
from jax import lax
from jax.experimental import pallas as pl
from jax.experimental.pallas import tpu as pltpu
from jax.experimental.pallas import tpu_sc as plsc


def _make_gather(V, D, B):
    info = plsc.get_sparse_core_info()
    NC, NS = info.num_cores, info.num_subcores
    NW = NC * NS
    assert B % (8 * NW) == 0 and V % 8 == 0
    b_per_w = B // NW
    mesh = plsc.VectorSubcoreMesh(core_axis_name="c", subcore_axis_name="s")

    @functools.partial(
        pl.kernel,
        mesh=mesh,
        out_type=jax.ShapeDtypeStruct((B, D), jnp.float32),
        scratch_types=[
            pltpu.VMEM((b_per_w,), jnp.int32),
            pltpu.VMEM((b_per_w, D), jnp.float32),
            pltpu.SemaphoreType.DMA,
        ],
        compiler_params=pltpu.CompilerParams(use_tc_tiling_on_sc=False),
    )
    def gather_kernel(y_hbm, table_hbm, out_hbm, y_v, rows_v, sem):
        wid = lax.axis_index("s") * NC + lax.axis_index("c")
        base = wid * b_per_w
        pltpu.sync_copy(y_hbm.at[pl.ds(base, b_per_w)], y_v)

        @pl.loop(0, b_per_w // 16, unroll=2)
        def _(k):
            vec = y_v[pl.ds(k * 16, 16)]
            t = vec >> 3
            s = vec & 7
            for j in range(16):
                pltpu.async_copy(
                    table_hbm.at[t[j], s[j]], rows_v.at[k * 16 + j], sem
                )

        dst = out_hbm.at[pl.ds(base, b_per_w)]
        pltpu.make_async_copy(dst, rows_v, sem).wait()
        pltpu.sync_copy(rows_v, dst)

    return gather_kernel


@jax.jit
def kernel(y, table):
    B, = y.shape
    V, D = table.shape
    table3 = table.reshape(V // 8, 8, D)
    return _make_gather(V, D, B)(y.astype(jnp.int32), table3)


# speedup vs baseline: 2.5977x; 2.5722x over previous
import functools

import jax
import jax.numpy as jnp
from jax import lax
from jax.experimental import pallas as pl
from jax.experimental.pallas import tpu as pltpu
from jax.experimental.pallas import tpu_sc as plsc


def _make_gather(V, D, B):
    info = plsc.get_sparse_core_info()
    NC, NS = info.num_cores, info.num_subcores
    NW = NC * NS
    assert B % (8 * NW) == 0 and V % 8 == 0
    b_per_w = B // NW
    mesh = plsc.VectorSubcoreMesh(core_axis_name="c", subcore_axis_name="s")

    @functools.partial(
        pl.kernel,
        mesh=mesh,
        out_type=jax.ShapeDtypeStruct((B, D), jnp.float32),
        scratch_types=[
            pltpu.VMEM((b_per_w,), jnp.int32),
            pltpu.VMEM((b_per_w, D), jnp.float32),
            pltpu.SemaphoreType.DMA,
        ],
    )
    def gather_kernel(y_hbm, table_hbm, out_hbm, y_v, rows_v, sem):
        wid = lax.axis_index("s") * NC + lax.axis_index("c")
        base = wid * b_per_w
        pltpu.sync_copy(y_hbm.at[pl.ds(base, b_per_w)], y_v)

        @pl.loop(0, b_per_w // 16, unroll=2)
        def _(k):
            vec = y_v[pl.ds(k * 16, 16)]
            t = vec >> 3
            s = vec & 7
            for j in range(16):
                pltpu.async_copy(
                    table_hbm.at[t[j], s[j]], rows_v.at[k * 16 + j], sem
                )

        dst = out_hbm.at[pl.ds(base, b_per_w)]
        pltpu.make_async_copy(dst, rows_v, sem).wait()
        pltpu.sync_copy(rows_v, dst)

    return gather_kernel


@jax.jit
def kernel(y, table):
    B, = y.shape
    V, D = table.shape
    table3 = table.reshape(V // 8, 8, D)
    return _make_gather(V, D, B)(y.astype(jnp.int32), table3)
